# Initial kernel scaffold; baseline (speedup 1.0000x reference)
import jax
import jax.numpy as jnp
from jax.experimental import pallas as pl


def kernel(score, target):
    raise NotImplementedError("write your pallas kernel here")


# speedup vs baseline: 29.5586x; 29.5586x over previous
import functools

import jax
import jax.numpy as jnp
from jax import lax
from jax.experimental import pallas as pl
from jax.experimental.pallas import tpu as pltpu

THR = 0.7
KEEP_MIN = 100000


def _ohem_body(score_ref, tgt_ref, p_ref, nll_ref, stats_ref, acc_ref):
    b = pl.program_id(0)
    i = pl.program_id(1)
    nb = pl.num_programs(0)
    ni = pl.num_programs(1)
    first = jnp.logical_and(b == 0, i == 0)
    last = jnp.logical_and(b == nb - 1, i == ni - 1)

    @pl.when(first)
    def _init():
        acc_ref[0] = jnp.float32(0.0)
        acc_ref[1] = jnp.float32(0.0)
        acc_ref[2] = jnp.float32(0.0)

    s = score_ref[0]
    t = tgt_ref[0]

    m = jnp.max(s, axis=0)
    e = jnp.exp(s - m[None])
    se = jnp.sum(e, axis=0)
    cls = lax.broadcasted_iota(jnp.int32, s.shape, 0)
    onehot = cls == t[None]
    e_t = jnp.sum(jnp.where(onehot, e, 0.0), axis=0)
    s_t = jnp.sum(jnp.where(onehot, s, 0.0), axis=0)

    p = e_t / se
    nll = jnp.log(se) - (s_t - m)

    p_ref[0] = p
    nll_ref[0] = nll

    lt = p < THR
    acc_ref[0] += jnp.sum(jnp.where(p <= THR, 1.0, 0.0))
    acc_ref[1] += jnp.sum(jnp.where(lt, 1.0, 0.0))
    acc_ref[2] += jnp.sum(jnp.where(lt, nll, 0.0))

    @pl.when(last)
    def _fin():
        stats_ref[0] = acc_ref[0]
        stats_ref[1] = acc_ref[1]
        stats_ref[2] = acc_ref[2]


def _ohem_pass(score, target):
    B, C, H, W = score.shape
    RH = 64
    grid = (B, H // RH)
    return pl.pallas_call(
        _ohem_body,
        grid=grid,
        in_specs=[
            pl.BlockSpec((1, C, RH, W), lambda b, i: (b, 0, i, 0)),
            pl.BlockSpec((1, RH, W), lambda b, i: (b, i, 0)),
        ],
        out_specs=[
            pl.BlockSpec((1, RH, W), lambda b, i: (b, i, 0)),
            pl.BlockSpec((1, RH, W), lambda b, i: (b, i, 0)),
            pl.BlockSpec(memory_space=pltpu.SMEM, index_map=lambda b, i: (0,)),
        ],
        out_shape=[
            jax.ShapeDtypeStruct((B, H, W), jnp.float32),
            jax.ShapeDtypeStruct((B, H, W), jnp.float32),
            jax.ShapeDtypeStruct((3,), jnp.float32),
        ],
        scratch_shapes=[pltpu.SMEM((3,), jnp.float32)],
    )(score, target)


def kernel(score, target):
    p, nll, stats = _ohem_pass(score, target)
    cnt_le, cnt_lt, loss_sum = stats[0], stats[1], stats[2]

    def common(_):
        return loss_sum / cnt_lt

    def rare(_):
        ps = jnp.sort(p.reshape(-1))
        thr = jnp.maximum(ps[KEEP_MIN - 1], THR)
        keep = p < thr
        tot = jnp.sum(jnp.where(keep, nll, 0.0))
        cnt = jnp.sum(keep).astype(jnp.float32)
        return tot / cnt

    return lax.cond(cnt_le >= KEEP_MIN, common, rare, None)
